# initial kernel scaffold (unmeasured)
import functools

import jax
import jax.numpy as jnp
import numpy as np
from jax import lax
from jax.experimental import pallas as pl
from jax.experimental.pallas import tpu as pltpu

N_DEV = 32
M_PER = 128
N_OUT = 2048


def _ring_tables():
    import distributed_mesh_v7x as dm

    mesh = dm.get_mesh("i", world_size=N_DEV)
    devs = list(mesh.devices.flat)
    coords = [tuple(getattr(d, "coords", (i, 0, 0))) for i, d in enumerate(devs)]
    logical_of = {c: i for i, c in enumerate(coords)}

    ring_coords = None
    if len(coords) == N_DEV and all(len(c) == 3 for c in coords):
        xs = sorted({c[0] for c in coords})
        ys = sorted({c[1] for c in coords})
        zs = sorted({c[2] for c in coords})
        if len(xs) == 2 and len(ys) * len(zs) == 16:
            path = []
            for zi, z in enumerate(zs):
                row = ys if zi % 2 == 0 else list(reversed(ys))
                path.extend((y, z) for y in row)
            ring_coords = [(xs[0], y, z) for (y, z) in path] + [
                (xs[1], y, z) for (y, z) in reversed(path)
            ]
            if set(ring_coords) != set(coords):
                ring_coords = None

    if ring_coords is None:
        ring_order = list(range(N_DEV))
    else:
        ring_order = [logical_of[c] for c in ring_coords]

    pos_of = [0] * N_DEV
    for p, l in enumerate(ring_order):
        pos_of[l] = p
    right_of = [ring_order[(pos_of[l] + 1) % N_DEV] for l in range(N_DEV)]
    left_of = [ring_order[(pos_of[l] - 1) % N_DEV] for l in range(N_DEV)]
    return (
        np.asarray(ring_order, np.int32),
        np.asarray(right_of, np.int32),
        np.asarray(left_of, np.int32),
        np.asarray(pos_of, np.int32),
    )


def kernel(x, w_mat):
    ring_order, right_of, left_of, pos_of = _ring_tables()

    def body(x_ref, w_ref, out_ref, xb_ref, wb_ref, send_buf, recv_buf,
             send_sems, recv_sems, credit_sem):
        my = lax.axis_index("i")
        ring_order_a = jnp.asarray(ring_order)
        right = jnp.asarray(right_of)[my]
        left = jnp.asarray(left_of)[my]
        my_pos = jnp.asarray(pos_of)[my]

        barrier_sem = pltpu.get_barrier_semaphore()
        for nbr in (left, right):
            pl.semaphore_signal(
                barrier_sem, inc=1,
                device_id=(nbr,), device_id_type=pl.DeviceIdType.MESH,
            )
        pl.semaphore_wait(barrier_sem, 2)

        xb_ref[...] = x_ref[...].astype(jnp.bfloat16)
        wb_ref[...] = w_ref[...].astype(jnp.bfloat16)

        def partial_for(dest_logical):
            xs = xb_ref[pl.ds(dest_logical * M_PER, M_PER), :]
            return jnp.dot(xs, wb_ref[...], preferred_element_type=jnp.float32)

        for t in range(N_DEV - 1):
            dest = ring_order_a[(my_pos - t - 1) % N_DEV]
            part = partial_for(dest)
            if t == 0:
                val = part
            else:
                val = part + recv_buf[(t - 1) % 2].astype(jnp.float32)
            send_buf[t % 2] = val.astype(jnp.bfloat16)

            if t >= 2:
                pl.semaphore_wait(credit_sem, 1)
            rdma = pltpu.make_async_remote_copy(
                src_ref=send_buf.at[t % 2],
                dst_ref=recv_buf.at[t % 2],
                send_sem=send_sems.at[t % 2],
                recv_sem=recv_sems.at[t % 2],
                device_id=(right,),
                device_id_type=pl.DeviceIdType.MESH,
            )
            rdma.start()
            if 1 <= t <= N_DEV - 3:
                pl.semaphore_signal(
                    credit_sem, inc=1,
                    device_id=(left,), device_id_type=pl.DeviceIdType.MESH,
                )
            rdma.wait()

        final = partial_for(my) + recv_buf[(N_DEV - 2) % 2].astype(jnp.float32)
        out_ref[...] = jnp.maximum(final, 0.0)

    m_tot, k_per = x.shape
    _, n_out = w_mat.shape
    return pl.pallas_call(
        body,
        out_shape=jax.ShapeDtypeStruct((M_PER, n_out), jnp.float32),
        in_specs=[
            pl.BlockSpec(memory_space=pltpu.VMEM),
            pl.BlockSpec(memory_space=pltpu.VMEM),
        ],
        out_specs=pl.BlockSpec(memory_space=pltpu.VMEM),
        scratch_shapes=[
            pltpu.VMEM((m_tot, k_per), jnp.bfloat16),
            pltpu.VMEM((k_per, n_out), jnp.bfloat16),
            pltpu.VMEM((2, M_PER, n_out), jnp.bfloat16),
            pltpu.VMEM((2, M_PER, n_out), jnp.bfloat16),
            pltpu.SemaphoreType.DMA((2,)),
            pltpu.SemaphoreType.DMA((2,)),
            pltpu.SemaphoreType.REGULAR,
        ],
        compiler_params=pltpu.CompilerParams(collective_id=0),
    )(x, w_mat)


# baseline (device time: 246636 ns/iter reference)
import jax
import jax.numpy as jnp
import numpy as np
from jax import lax
from jax.experimental import pallas as pl
from jax.experimental.pallas import tpu as pltpu

N_DEV = 32
M_PER = 128


def _ring_tables():
    import distributed_mesh_v7x as dm

    mesh = dm.get_mesh("i", world_size=N_DEV)
    devs = list(mesh.devices.flat)
    coords = [tuple(getattr(d, "coords", (i, 0, 0))) for i, d in enumerate(devs)]
    logical_of = {c: i for i, c in enumerate(coords)}

    ring_coords = None
    if len(set(coords)) == N_DEV and all(len(c) == 3 for c in coords):
        xs = sorted({c[0] for c in coords})
        ys = sorted({c[1] for c in coords})
        zs = sorted({c[2] for c in coords})
        if len(xs) == 2 and len(ys) * len(zs) == N_DEV // 2:
            path = []
            for zi, z in enumerate(zs):
                row = ys if zi % 2 == 0 else list(reversed(ys))
                path.extend((y, z) for y in row)
            ring_coords = [(xs[0], y, z) for (y, z) in path] + [
                (xs[1], y, z) for (y, z) in reversed(path)
            ]
            if set(ring_coords) != set(coords):
                ring_coords = None

    if ring_coords is None:
        ring_order = list(range(N_DEV))
    else:
        ring_order = [logical_of[c] for c in ring_coords]

    pos_of = [0] * N_DEV
    for p, l in enumerate(ring_order):
        pos_of[l] = p
    right_of = [ring_order[(pos_of[l] + 1) % N_DEV] for l in range(N_DEV)]
    left_of = [ring_order[(pos_of[l] - 1) % N_DEV] for l in range(N_DEV)]
    return (
        np.asarray(ring_order, np.int32),
        np.asarray(right_of, np.int32),
        np.asarray(left_of, np.int32),
        np.asarray(pos_of, np.int32),
    )


def kernel(x, w_mat):
    ring_order, right_of, left_of, pos_of = _ring_tables()

    def body(x_ref, w_ref, ring_ref, right_ref, left_ref, pos_ref, out_ref,
             xb_ref, wb_ref, send_buf, recv_buf, send_sems, recv_sems):
        my = lax.axis_index("i")
        right = right_ref[my]
        left = left_ref[my]
        my_pos = pos_ref[my]

        barrier_sem = pltpu.get_barrier_semaphore()
        for nbr in (left, right):
            pl.semaphore_signal(
                barrier_sem, inc=1,
                device_id=(nbr,), device_id_type=pl.DeviceIdType.MESH,
            )
        pl.semaphore_wait(barrier_sem, 2)

        xb_ref[...] = x_ref[...].astype(jnp.bfloat16)
        wb_ref[...] = w_ref[...].astype(jnp.bfloat16)

        def partial_for(dest_logical):
            xs = xb_ref[pl.ds(dest_logical * M_PER, M_PER), :]
            return jnp.dot(xs, wb_ref[...], preferred_element_type=jnp.float32)

        for t in range(N_DEV - 1):
            dest = ring_ref[(my_pos - (t + 1)) % N_DEV]
            part = partial_for(dest)
            if t == 0:
                val = part
            else:
                val = part + recv_buf[t - 1].astype(jnp.float32)
            send_buf[t % 2] = val.astype(jnp.bfloat16)

            rdma = pltpu.make_async_remote_copy(
                src_ref=send_buf.at[t % 2],
                dst_ref=recv_buf.at[t],
                send_sem=send_sems.at[t % 2],
                recv_sem=recv_sems.at[t],
                device_id=(right,),
                device_id_type=pl.DeviceIdType.MESH,
            )
            rdma.start()
            rdma.wait()

        final = partial_for(my) + recv_buf[N_DEV - 2].astype(jnp.float32)
        out_ref[...] = jnp.maximum(final, 0.0)

    m_tot, k_per = x.shape
    _, n_out = w_mat.shape
    return pl.pallas_call(
        body,
        out_shape=jax.ShapeDtypeStruct((M_PER, n_out), jnp.float32),
        in_specs=[
            pl.BlockSpec(memory_space=pltpu.VMEM),
            pl.BlockSpec(memory_space=pltpu.VMEM),
            pl.BlockSpec(memory_space=pltpu.SMEM),
            pl.BlockSpec(memory_space=pltpu.SMEM),
            pl.BlockSpec(memory_space=pltpu.SMEM),
            pl.BlockSpec(memory_space=pltpu.SMEM),
        ],
        out_specs=pl.BlockSpec(memory_space=pltpu.VMEM),
        scratch_shapes=[
            pltpu.VMEM((m_tot, k_per), jnp.bfloat16),
            pltpu.VMEM((k_per, n_out), jnp.bfloat16),
            pltpu.VMEM((2, M_PER, n_out), jnp.bfloat16),
            pltpu.VMEM((N_DEV - 1, M_PER, n_out), jnp.bfloat16),
            pltpu.SemaphoreType.DMA((2,)),
            pltpu.SemaphoreType.DMA((N_DEV - 1,)),
        ],
        compiler_params=pltpu.CompilerParams(collective_id=0),
    )(
        x, w_mat,
        jnp.asarray(ring_order), jnp.asarray(right_of),
        jnp.asarray(left_of), jnp.asarray(pos_of),
    )


# device time: 112589 ns/iter; 2.1906x vs baseline; 2.1906x over previous
import jax
import jax.numpy as jnp
import numpy as np
from jax import lax
from jax.experimental import pallas as pl
from jax.experimental.pallas import tpu as pltpu

N_DEV = 32
M_PER = 128
HALF = 1024
SUBS = 2
CW = HALF // SUBS


def _ring_tables():
    import distributed_mesh_v7x as dm

    mesh = dm.get_mesh("i", world_size=N_DEV)
    devs = list(mesh.devices.flat)
    coords = [tuple(getattr(d, "coords", (i, 0, 0))) for i, d in enumerate(devs)]
    logical_of = {c: i for i, c in enumerate(coords)}

    ring_coords = None
    if len(set(coords)) == N_DEV and all(len(c) == 3 for c in coords):
        xs = sorted({c[0] for c in coords})
        ys = sorted({c[1] for c in coords})
        zs = sorted({c[2] for c in coords})
        if len(xs) == 2 and len(ys) * len(zs) == N_DEV // 2:
            path = []
            for zi, z in enumerate(zs):
                row = ys if zi % 2 == 0 else list(reversed(ys))
                path.extend((y, z) for y in row)
            ring_coords = [(xs[0], y, z) for (y, z) in path] + [
                (xs[1], y, z) for (y, z) in reversed(path)
            ]
            if set(ring_coords) != set(coords):
                ring_coords = None

    if ring_coords is None:
        ring_order = list(range(N_DEV))
    else:
        ring_order = [logical_of[c] for c in ring_coords]

    pos_of = [0] * N_DEV
    for p, l in enumerate(ring_order):
        pos_of[l] = p
    right_of = [ring_order[(pos_of[l] + 1) % N_DEV] for l in range(N_DEV)]
    left_of = [ring_order[(pos_of[l] - 1) % N_DEV] for l in range(N_DEV)]
    return (
        np.asarray(ring_order, np.int32),
        np.asarray(right_of, np.int32),
        np.asarray(left_of, np.int32),
        np.asarray(pos_of, np.int32),
    )


def kernel(x, w_mat):
    ring_order, right_of, left_of, pos_of = _ring_tables()

    def body(x_ref, w_ref, ring_ref, right_ref, left_ref, pos_ref, out_ref,
             xb_ref, wb_ref, sbufR, sbufL, rbufR, rbufL,
             ssemR, ssemL, rsemR, rsemL):
        my = lax.axis_index("i")
        right = right_ref[my]
        left = left_ref[my]
        my_pos = pos_ref[my]

        barrier_sem = pltpu.get_barrier_semaphore()
        for nbr in (left, right):
            pl.semaphore_signal(
                barrier_sem, inc=1,
                device_id=(nbr,), device_id_type=pl.DeviceIdType.MESH,
            )
        pl.semaphore_wait(barrier_sem, 2)

        xb_ref[...] = x_ref[...].astype(jnp.bfloat16)
        wb_ref[...] = w_ref[...].astype(jnp.bfloat16)

        def partial(dest_logical, lo):
            xs = xb_ref[pl.ds(dest_logical * M_PER, M_PER), :]
            return jnp.dot(xs, wb_ref[:, lo:lo + HALF],
                           preferred_element_type=jnp.float32)

        def desc(sbuf, rbuf, ssem, rsem, t, s, tgt):
            c0 = s * CW
            return pltpu.make_async_remote_copy(
                src_ref=sbuf.at[t % 2, :, c0:c0 + CW],
                dst_ref=rbuf.at[t, :, c0:c0 + CW],
                send_sem=ssem.at[(t % 2) * SUBS + s],
                recv_sem=rsem.at[t * SUBS + s],
                device_id=(tgt,),
                device_id_type=pl.DeviceIdType.MESH,
            )

        for t in range(N_DEV - 1):
            destR = ring_ref[(my_pos - (t + 1)) % N_DEV]
            destL = ring_ref[(my_pos + (t + 1)) % N_DEV]
            partR = partial(destR, 0)
            partL = partial(destL, HALF)
            for s in range(SUBS):
                c0 = s * CW
                for part, sbuf, rbuf, ssem, rsem, tgt in (
                    (partR, sbufR, rbufR, ssemR, rsemR, right),
                    (partL, sbufL, rbufL, ssemL, rsemL, left),
                ):
                    val = part[:, c0:c0 + CW]
                    if t >= 1:
                        desc(sbuf, rbuf, ssem, rsem, t - 1, s, tgt).wait_recv()
                        val = val + rbuf[t - 1, :, c0:c0 + CW].astype(jnp.float32)
                    if t >= 2:
                        desc(sbuf, rbuf, ssem, rsem, t - 2, s, tgt).wait_send()
                    sbuf[t % 2, :, c0:c0 + CW] = val.astype(jnp.bfloat16)
                    desc(sbuf, rbuf, ssem, rsem, t, s, tgt).start()

        fR = partial(my, 0)
        fL = partial(my, HALF)
        for s in range(SUBS):
            desc(sbufR, rbufR, ssemR, rsemR, N_DEV - 2, s, right).wait_recv()
            desc(sbufL, rbufL, ssemL, rsemL, N_DEV - 2, s, left).wait_recv()
        out_ref[:, 0:HALF] = jnp.maximum(
            fR + rbufR[N_DEV - 2].astype(jnp.float32), 0.0)
        out_ref[:, HALF:2 * HALF] = jnp.maximum(
            fL + rbufL[N_DEV - 2].astype(jnp.float32), 0.0)
        for s in range(SUBS):
            for t in (N_DEV - 3, N_DEV - 2):
                desc(sbufR, rbufR, ssemR, rsemR, t, s, right).wait_send()
                desc(sbufL, rbufL, ssemL, rsemL, t, s, left).wait_send()

    m_tot, k_per = x.shape
    _, n_out = w_mat.shape
    return pl.pallas_call(
        body,
        out_shape=jax.ShapeDtypeStruct((M_PER, n_out), jnp.float32),
        in_specs=[
            pl.BlockSpec(memory_space=pltpu.VMEM),
            pl.BlockSpec(memory_space=pltpu.VMEM),
            pl.BlockSpec(memory_space=pltpu.SMEM),
            pl.BlockSpec(memory_space=pltpu.SMEM),
            pl.BlockSpec(memory_space=pltpu.SMEM),
            pl.BlockSpec(memory_space=pltpu.SMEM),
        ],
        out_specs=pl.BlockSpec(memory_space=pltpu.VMEM),
        scratch_shapes=[
            pltpu.VMEM((m_tot, k_per), jnp.bfloat16),
            pltpu.VMEM((k_per, n_out), jnp.bfloat16),
            pltpu.VMEM((2, M_PER, HALF), jnp.bfloat16),
            pltpu.VMEM((2, M_PER, HALF), jnp.bfloat16),
            pltpu.VMEM((N_DEV - 1, M_PER, HALF), jnp.bfloat16),
            pltpu.VMEM((N_DEV - 1, M_PER, HALF), jnp.bfloat16),
            pltpu.SemaphoreType.DMA((2 * SUBS,)),
            pltpu.SemaphoreType.DMA((2 * SUBS,)),
            pltpu.SemaphoreType.DMA(((N_DEV - 1) * SUBS,)),
            pltpu.SemaphoreType.DMA(((N_DEV - 1) * SUBS,)),
        ],
        compiler_params=pltpu.CompilerParams(collective_id=0),
    )(
        x, w_mat,
        jnp.asarray(ring_order), jnp.asarray(right_of),
        jnp.asarray(left_of), jnp.asarray(pos_of),
    )


# device time: 109538 ns/iter; 2.2516x vs baseline; 1.0279x over previous
import jax
import jax.numpy as jnp
import numpy as np
from jax import lax
from jax.experimental import pallas as pl
from jax.experimental.pallas import tpu as pltpu


N_DEV = 32
M_PER = 128
HALF = 1024
SUBS = 2
CW = HALF // SUBS


def _ring_tables():
    import distributed_mesh_v7x as dm

    mesh = dm.get_mesh("i", world_size=N_DEV)
    devs = list(mesh.devices.flat)
    coords = [tuple(getattr(d, "coords", (i, 0, 0))) for i, d in enumerate(devs)]
    logical_of = {c: i for i, c in enumerate(coords)}

    ring_coords = None
    if len(set(coords)) == N_DEV and all(len(c) == 3 for c in coords):
        xs = sorted({c[0] for c in coords})
        ys = sorted({c[1] for c in coords})
        zs = sorted({c[2] for c in coords})
        if len(xs) == 2 and len(ys) * len(zs) == N_DEV // 2:
            path = []
            for zi, z in enumerate(zs):
                row = ys if zi % 2 == 0 else list(reversed(ys))
                path.extend((y, z) for y in row)
            ring_coords = [(xs[0], y, z) for (y, z) in path] + [
                (xs[1], y, z) for (y, z) in reversed(path)
            ]
            if set(ring_coords) != set(coords):
                ring_coords = None

    if ring_coords is None:
        ring_order = list(range(N_DEV))
    else:
        ring_order = [logical_of[c] for c in ring_coords]

    pos_of = [0] * N_DEV
    for p, l in enumerate(ring_order):
        pos_of[l] = p
    right_of = [ring_order[(pos_of[l] + 1) % N_DEV] for l in range(N_DEV)]
    left_of = [ring_order[(pos_of[l] - 1) % N_DEV] for l in range(N_DEV)]
    return (
        np.asarray(ring_order, np.int32),
        np.asarray(right_of, np.int32),
        np.asarray(left_of, np.int32),
        np.asarray(pos_of, np.int32),
    )


def kernel(x, w_mat):
    ring_order, right_of, left_of, pos_of = _ring_tables()

    def body(x_ref, w_ref, ring_ref, right_ref, left_ref, pos_ref, out_ref,
             sbufR, sbufL, rbufR, rbufL,
             ssemR, ssemL, rsemR, rsemL):
        my = lax.axis_index("i")
        right = right_ref[my]
        left = left_ref[my]
        my_pos = pos_ref[my]

        barrier_sem = pltpu.get_barrier_semaphore()
        for nbr in (left, right):
            pl.semaphore_signal(
                barrier_sem, inc=1,
                device_id=(nbr,), device_id_type=pl.DeviceIdType.MESH,
            )
        pl.semaphore_wait(barrier_sem, 2)

        def partial(dest_logical, lo):
            xs = x_ref[pl.ds(dest_logical * M_PER, M_PER), :]
            return jnp.dot(xs, w_ref[:, lo:lo + HALF],
                           preferred_element_type=jnp.float32
                           ).astype(jnp.bfloat16)

        def desc(sbuf, rbuf, ssem, rsem, slot, rt, s, tgt):
            c0 = s * CW
            return pltpu.make_async_remote_copy(
                src_ref=sbuf.at[slot, :, c0:c0 + CW],
                dst_ref=rbuf.at[rt, :, c0:c0 + CW],
                send_sem=ssem.at[slot * SUBS + s],
                recv_sem=rsem.at[rt * SUBS + s],
                device_id=(tgt,),
                device_id_type=pl.DeviceIdType.MESH,
            )

        def step(t, do_recv, do_send_wait):
            partR = partial(ring_ref[(my_pos - (t + 1)) % N_DEV], 0)
            partL = partial(ring_ref[(my_pos + (t + 1)) % N_DEV], HALF)
            slot = t % 2
            for s in range(SUBS):
                c0 = s * CW
                for part, sbuf, rbuf, ssem, rsem, tgt, off in (
                    (partR, sbufR, rbufR, ssemR, rsemR, right, 0),
                    (partL, sbufL, rbufL, ssemL, rsemL, left, HALF),
                ):
                    val = part[:, c0:c0 + CW]
                    if do_recv:
                        desc(sbuf, rbuf, ssem, rsem, 1 - slot, t - 1, s,
                             tgt).wait_recv()
                        val = val + rbuf[t - 1, :, c0:c0 + CW]
                    if do_send_wait:
                        desc(sbuf, rbuf, ssem, rsem, slot, t - 2, s,
                             tgt).wait_send()
                    sbuf[slot, :, c0:c0 + CW] = val
                    desc(sbuf, rbuf, ssem, rsem, slot, t, s, tgt).start()

        for t in range(N_DEV - 1):
            step(t, t >= 1, t >= 2)

        fR = partial(my, 0)
        fL = partial(my, HALF)
        for s in range(SUBS):
            desc(sbufR, rbufR, ssemR, rsemR, (N_DEV - 2) % 2, N_DEV - 2, s,
                 right).wait_recv()
            desc(sbufL, rbufL, ssemL, rsemL, (N_DEV - 2) % 2, N_DEV - 2, s,
                 left).wait_recv()
        out_ref[:, 0:HALF] = jnp.maximum(rbufR[N_DEV - 2] + fR, 0.0)
        out_ref[:, HALF:2 * HALF] = jnp.maximum(rbufL[N_DEV - 2] + fL, 0.0)
        for s in range(SUBS):
            for t in (N_DEV - 3, N_DEV - 2):
                desc(sbufR, rbufR, ssemR, rsemR, t % 2, t, s, right).wait_send()
                desc(sbufL, rbufL, ssemL, rsemL, t % 2, t, s, left).wait_send()

    m_tot, k_per = x.shape
    _, n_out = w_mat.shape
    return pl.pallas_call(
        body,
        out_shape=jax.ShapeDtypeStruct((M_PER, n_out), jnp.bfloat16),
        in_specs=[
            pl.BlockSpec(memory_space=pltpu.VMEM),
            pl.BlockSpec(memory_space=pltpu.VMEM),
            pl.BlockSpec(memory_space=pltpu.SMEM),
            pl.BlockSpec(memory_space=pltpu.SMEM),
            pl.BlockSpec(memory_space=pltpu.SMEM),
            pl.BlockSpec(memory_space=pltpu.SMEM),
        ],
        out_specs=pl.BlockSpec(memory_space=pltpu.VMEM),
        scratch_shapes=[
            pltpu.VMEM((2, M_PER, HALF), jnp.bfloat16),
            pltpu.VMEM((2, M_PER, HALF), jnp.bfloat16),
            pltpu.VMEM((N_DEV - 1, M_PER, HALF), jnp.bfloat16),
            pltpu.VMEM((N_DEV - 1, M_PER, HALF), jnp.bfloat16),
            pltpu.SemaphoreType.DMA((2 * SUBS,)),
            pltpu.SemaphoreType.DMA((2 * SUBS,)),
            pltpu.SemaphoreType.DMA(((N_DEV - 1) * SUBS,)),
            pltpu.SemaphoreType.DMA(((N_DEV - 1) * SUBS,)),
        ],
        compiler_params=pltpu.CompilerParams(collective_id=0),
    )(
        x.astype(jnp.bfloat16), w_mat.astype(jnp.bfloat16),
        jnp.asarray(ring_order), jnp.asarray(right_of),
        jnp.asarray(left_of), jnp.asarray(pos_of),
    )


# device time: 99727 ns/iter; 2.4731x vs baseline; 1.0984x over previous
import jax
import jax.numpy as jnp
import numpy as np
from jax import lax
from jax.experimental import pallas as pl
from jax.experimental.pallas import tpu as pltpu


N_DEV = 32
M_PER = 128
HALF = 1024
SUBS = 4
CW = HALF // SUBS


def _ring_tables():
    import distributed_mesh_v7x as dm

    mesh = dm.get_mesh("i", world_size=N_DEV)
    devs = list(mesh.devices.flat)
    coords = [tuple(getattr(d, "coords", (i, 0, 0))) for i, d in enumerate(devs)]
    logical_of = {c: i for i, c in enumerate(coords)}

    ring_coords = None
    if len(set(coords)) == N_DEV and all(len(c) == 3 for c in coords):
        xs = sorted({c[0] for c in coords})
        ys = sorted({c[1] for c in coords})
        zs = sorted({c[2] for c in coords})
        if len(xs) == 2 and len(ys) * len(zs) == N_DEV // 2:
            path = []
            for zi, z in enumerate(zs):
                row = ys if zi % 2 == 0 else list(reversed(ys))
                path.extend((y, z) for y in row)
            ring_coords = [(xs[0], y, z) for (y, z) in path] + [
                (xs[1], y, z) for (y, z) in reversed(path)
            ]
            if set(ring_coords) != set(coords):
                ring_coords = None

    if ring_coords is None:
        ring_order = list(range(N_DEV))
    else:
        ring_order = [logical_of[c] for c in ring_coords]

    pos_of = [0] * N_DEV
    for p, l in enumerate(ring_order):
        pos_of[l] = p
    right_of = [ring_order[(pos_of[l] + 1) % N_DEV] for l in range(N_DEV)]
    left_of = [ring_order[(pos_of[l] - 1) % N_DEV] for l in range(N_DEV)]
    return (
        np.asarray(ring_order, np.int32),
        np.asarray(right_of, np.int32),
        np.asarray(left_of, np.int32),
        np.asarray(pos_of, np.int32),
    )


def kernel(x, w_mat):
    ring_order, right_of, left_of, pos_of = _ring_tables()

    def body(x_ref, w_ref, ring_ref, right_ref, left_ref, pos_ref, out_ref,
             sbufR, sbufL, rbufR, rbufL,
             ssemR, ssemL, rsemR, rsemL):
        my = lax.axis_index("i")
        right = right_ref[my]
        left = left_ref[my]
        my_pos = pos_ref[my]

        barrier_sem = pltpu.get_barrier_semaphore()
        for nbr in (left, right):
            pl.semaphore_signal(
                barrier_sem, inc=1,
                device_id=(nbr,), device_id_type=pl.DeviceIdType.MESH,
            )
        pl.semaphore_wait(barrier_sem, 2)

        def partial(dest_logical, lo):
            xs = x_ref[pl.ds(dest_logical * M_PER, M_PER), :]
            return jnp.dot(xs, w_ref[:, lo:lo + HALF],
                           preferred_element_type=jnp.float32
                           ).astype(jnp.bfloat16)

        def desc(sbuf, rbuf, ssem, rsem, slot, rt, s, tgt):
            c0 = s * CW
            return pltpu.make_async_remote_copy(
                src_ref=sbuf.at[slot, :, c0:c0 + CW],
                dst_ref=rbuf.at[rt, :, c0:c0 + CW],
                send_sem=ssem.at[slot * SUBS + s],
                recv_sem=rsem.at[rt * SUBS + s],
                device_id=(tgt,),
                device_id_type=pl.DeviceIdType.MESH,
            )

        def step(t, do_recv, do_send_wait):
            partR = partial(ring_ref[(my_pos - (t + 1)) % N_DEV], 0)
            partL = partial(ring_ref[(my_pos + (t + 1)) % N_DEV], HALF)
            slot = t % 2
            for s in range(SUBS):
                c0 = s * CW
                for part, sbuf, rbuf, ssem, rsem, tgt, off in (
                    (partR, sbufR, rbufR, ssemR, rsemR, right, 0),
                    (partL, sbufL, rbufL, ssemL, rsemL, left, HALF),
                ):
                    val = part[:, c0:c0 + CW]
                    if do_recv:
                        desc(sbuf, rbuf, ssem, rsem, 1 - slot, t - 1, s,
                             tgt).wait_recv()
                        val = val + rbuf[t - 1, :, c0:c0 + CW]
                    if do_send_wait:
                        desc(sbuf, rbuf, ssem, rsem, slot, t - 2, s,
                             tgt).wait_send()
                    sbuf[slot, :, c0:c0 + CW] = val
                    desc(sbuf, rbuf, ssem, rsem, slot, t, s, tgt).start()

        for t in range(N_DEV - 1):
            step(t, t >= 1, t >= 2)

        fR = partial(my, 0)
        fL = partial(my, HALF)
        for s in range(SUBS):
            desc(sbufR, rbufR, ssemR, rsemR, (N_DEV - 2) % 2, N_DEV - 2, s,
                 right).wait_recv()
            desc(sbufL, rbufL, ssemL, rsemL, (N_DEV - 2) % 2, N_DEV - 2, s,
                 left).wait_recv()
        out_ref[:, 0:HALF] = jnp.maximum(rbufR[N_DEV - 2] + fR, 0.0)
        out_ref[:, HALF:2 * HALF] = jnp.maximum(rbufL[N_DEV - 2] + fL, 0.0)
        for s in range(SUBS):
            for t in (N_DEV - 3, N_DEV - 2):
                desc(sbufR, rbufR, ssemR, rsemR, t % 2, t, s, right).wait_send()
                desc(sbufL, rbufL, ssemL, rsemL, t % 2, t, s, left).wait_send()

    m_tot, k_per = x.shape
    _, n_out = w_mat.shape
    return pl.pallas_call(
        body,
        out_shape=jax.ShapeDtypeStruct((M_PER, n_out), jnp.bfloat16),
        in_specs=[
            pl.BlockSpec(memory_space=pltpu.VMEM),
            pl.BlockSpec(memory_space=pltpu.VMEM),
            pl.BlockSpec(memory_space=pltpu.SMEM),
            pl.BlockSpec(memory_space=pltpu.SMEM),
            pl.BlockSpec(memory_space=pltpu.SMEM),
            pl.BlockSpec(memory_space=pltpu.SMEM),
        ],
        out_specs=pl.BlockSpec(memory_space=pltpu.VMEM),
        scratch_shapes=[
            pltpu.VMEM((2, M_PER, HALF), jnp.bfloat16),
            pltpu.VMEM((2, M_PER, HALF), jnp.bfloat16),
            pltpu.VMEM((N_DEV - 1, M_PER, HALF), jnp.bfloat16),
            pltpu.VMEM((N_DEV - 1, M_PER, HALF), jnp.bfloat16),
            pltpu.SemaphoreType.DMA((2 * SUBS,)),
            pltpu.SemaphoreType.DMA((2 * SUBS,)),
            pltpu.SemaphoreType.DMA(((N_DEV - 1) * SUBS,)),
            pltpu.SemaphoreType.DMA(((N_DEV - 1) * SUBS,)),
        ],
        compiler_params=pltpu.CompilerParams(collective_id=0),
    )(
        x.astype(jnp.bfloat16), w_mat.astype(jnp.bfloat16),
        jnp.asarray(ring_order), jnp.asarray(right_of),
        jnp.asarray(left_of), jnp.asarray(pos_of),
    )
